# baseline (speedup 1.0000x reference)
import functools

import jax
import jax.numpy as jnp
from jax import lax
from jax.experimental import pallas as pl
from jax.experimental.pallas import tpu as pltpu
from jax.experimental.pallas import tpu_sc as plsc

B = 16384
D = 64
NUM_CORES = 2
NUM_SUBCORES = 16
NW = NUM_CORES * NUM_SUBCORES
BPW = B // NW
CHUNK = 128
NCHUNK = BPW // CHUNK


def _sc_gather(user_table2, item_table2, uidx2d, iidx2d):
    mesh = plsc.VectorSubcoreMesh(
        core_axis_name="c", subcore_axis_name="s",
        num_cores=NUM_CORES, num_subcores=NUM_SUBCORES,
    )

    @functools.partial(
        pl.kernel,
        out_type=(
            jax.ShapeDtypeStruct((B, 2 * D), jnp.float32),
            jax.ShapeDtypeStruct((B, 2 * D), jnp.float32),
        ),
        mesh=mesh,
        scratch_types=[
            pltpu.VMEM((NCHUNK, CHUNK), jnp.int32),
            pltpu.VMEM((NCHUNK, CHUNK), jnp.int32),
            pltpu.VMEM((BPW, 2 * D), jnp.float32),
            pltpu.SemaphoreType.DMA,
        ],
    )
    def gather_kernel(u_tab, i_tab, u_idx, i_idx, out_u, out_i,
                      uidx_v, iidx_v, rows_v, sem):
        wid = lax.axis_index("s") * NUM_CORES + lax.axis_index("c")
        base = wid * BPW
        crow = wid * NCHUNK
        pltpu.sync_copy(u_idx.at[pl.ds(crow, NCHUNK)], uidx_v)
        pltpu.sync_copy(i_idx.at[pl.ds(crow, NCHUNK)], iidx_v)
        for tab, idx_v, out in ((u_tab, uidx_v, out_u),
                                (i_tab, iidx_v, out_i)):
            copies = []
            for j in range(NCHUNK):
                copies.append(pltpu.async_copy(
                    tab.at[idx_v.at[j]],
                    rows_v.at[pl.ds(j * CHUNK, CHUNK)], sem))
            for c in copies:
                c.wait()
            pltpu.sync_copy(rows_v, out.at[pl.ds(base, BPW)])

    return gather_kernel(user_table2, item_table2, uidx2d, iidx2d)


def _combine_body(u_ref, v_ref, up_ref, vp_ref, o_ref):
    u = jnp.where(up_ref[...] == 0, u_ref[:, :D], u_ref[:, D:])
    v = jnp.where(vp_ref[...] == 0, v_ref[:, :D], v_ref[:, D:])
    mu = jnp.mean(u, axis=0, keepdims=True)
    mv = jnp.mean(v, axis=0, keepdims=True)
    fu = u + 7.0 * mu
    fv = v + 7.0 * mv
    inter = jnp.sum(fu * fv, axis=1, keepdims=True) * (1.0 / 16.0)
    o_ref[...] = jax.nn.sigmoid(inter)


def kernel(user_indices, item_indices, user_table, item_table):
    ut2 = user_table.reshape(-1, 2 * D)
    it2 = item_table.reshape(-1, 2 * D)
    urow = jax.lax.shift_right_logical(user_indices, 1)
    irow = jax.lax.shift_right_logical(item_indices, 1)
    upar = jnp.bitwise_and(user_indices, 1).reshape(B, 1)
    ipar = jnp.bitwise_and(item_indices, 1).reshape(B, 1)
    u_rows, i_rows = _sc_gather(ut2, it2,
                                urow.reshape(B // CHUNK, CHUNK),
                                irow.reshape(B // CHUNK, CHUNK))
    out = pl.pallas_call(
        _combine_body,
        out_shape=jax.ShapeDtypeStruct((B, 1), jnp.float32),
        compiler_params=pltpu.CompilerParams(
            vmem_limit_bytes=100 * 1024 * 1024),
    )(u_rows, i_rows, upar, ipar)
    return out.reshape(B)


# speedup vs baseline: 2.2253x; 2.2253x over previous
import functools

import jax
import jax.numpy as jnp
from jax import lax
from jax.experimental import pallas as pl
from jax.experimental.pallas import tpu as pltpu
from jax.experimental.pallas import tpu_sc as plsc

B = 16384
D = 64
NUM_CORES = 2
NUM_SUBCORES = 16
NW = NUM_CORES * NUM_SUBCORES
BPW = B // NW
NB = 8
GW = 128


def _sc_gather(u_tab_t, i_tab_t, user_indices, item_indices):
    mesh = plsc.VectorSubcoreMesh(
        core_axis_name="c", subcore_axis_name="s",
        num_cores=NUM_CORES, num_subcores=NUM_SUBCORES,
    )

    @functools.partial(
        pl.kernel,
        out_type=(
            jax.ShapeDtypeStruct((B, D), jnp.float32),
            jax.ShapeDtypeStruct((B, D), jnp.float32),
        ),
        mesh=mesh,
        compiler_params=pltpu.CompilerParams(
            use_tc_tiling_on_sc=True, needs_layout_passes=False),
        scratch_types=[
            pltpu.VMEM((BPW,), jnp.int32),
            pltpu.VMEM((NB, D, GW), jnp.float32),
            pltpu.VMEM((BPW // 2, D), jnp.float32),
            pltpu.SemaphoreType.DMA,
        ],
    )
    def gather_kernel(u_tab, i_tab, u_idx, i_idx, out_u, out_i,
                      idx_v, blk, rows, sem):
        wid = lax.axis_index("s") * NUM_CORES + lax.axis_index("c")
        base = wid * BPW
        krow = lax.iota(jnp.int32, 16)

        def extract(b, col, r):
            cvec = jnp.full((16,), col, dtype=jnp.int32)
            for q in range(D // 16):
                vals = plsc.load_gather(blk.at[b], [krow + 16 * q, cvec])
                rows[r, pl.ds(16 * q, 16)] = vals

        def make_body(tab):
            def body(t, carry):
                i0 = t * 16
                vec = idx_v[pl.ds(i0, 16)]
                col_vec = vec & (GW - 1)
                r0 = (t % (BPW // 32)) * 16
                for w in range(2):
                    copies = []
                    for b in range(NB):
                        l = w * NB + b
                        off = pl.multiple_of(vec[l] & ~(GW - 1), GW)
                        copies.append(pltpu.async_copy(
                            tab.at[:, pl.ds(off, GW)], blk.at[b], sem))
                    for c in copies:
                        c.wait()
                    for b in range(NB):
                        l = w * NB + b
                        extract(b, col_vec[l], r0 + l)
                return carry
            return body

        half = BPW // 2
        for tab, idx, out in ((u_tab, u_idx, out_u), (i_tab, i_idx, out_i)):
            pltpu.sync_copy(idx.at[pl.ds(base, BPW)], idx_v)
            body = make_body(tab)
            for h in range(2):
                lax.fori_loop(h * (half // 16), (h + 1) * (half // 16),
                              body, 0)
                pltpu.sync_copy(rows, out.at[pl.ds(base + h * half, half)])

    return gather_kernel(u_tab_t, i_tab_t, user_indices, item_indices)


def _combine_body(u_ref, v_ref, o_ref):
    u = u_ref[...]
    v = v_ref[...]
    mu = jnp.mean(u, axis=0, keepdims=True)
    mv = jnp.mean(v, axis=0, keepdims=True)
    fu = u + 7.0 * mu
    fv = v + 7.0 * mv
    inter = jnp.sum(fu * fv, axis=1, keepdims=True) * (1.0 / 16.0)
    o_ref[...] = jax.nn.sigmoid(inter)


def kernel(user_indices, item_indices, user_table, item_table):
    u_rows, i_rows = _sc_gather(user_table.T, item_table.T,
                                user_indices, item_indices)
    out = pl.pallas_call(
        _combine_body,
        out_shape=jax.ShapeDtypeStruct((B, 1), jnp.float32),
        compiler_params=pltpu.CompilerParams(
            vmem_limit_bytes=100 * 1024 * 1024),
    )(u_rows, i_rows)
    return out.reshape(B)


# speedup vs baseline: 3.2816x; 1.4746x over previous
import functools

import jax
import jax.numpy as jnp
from jax import lax
from jax.experimental import pallas as pl
from jax.experimental.pallas import tpu as pltpu
from jax.experimental.pallas import tpu_sc as plsc

B = 16384
D = 64
NUM_CORES = 2
NUM_SUBCORES = 16
NW = NUM_CORES * NUM_SUBCORES
BPW = B // NW
GW = 128
GS = 4
NGRP = BPW // GS
QGRP = NGRP // 4
QROWS = BPW // 4


def _sc_gather(u_tab_t, i_tab_t, su, pu2d, si, pi2d):
    mesh = plsc.VectorSubcoreMesh(
        core_axis_name="c", subcore_axis_name="s",
        num_cores=NUM_CORES, num_subcores=NUM_SUBCORES,
    )

    @functools.partial(
        pl.kernel,
        out_type=(
            jax.ShapeDtypeStruct((B, GW), jnp.float32),
            jax.ShapeDtypeStruct((B, GW), jnp.float32),
        ),
        mesh=mesh,
        compiler_params=pltpu.CompilerParams(
            use_tc_tiling_on_sc=True, needs_layout_passes=False),
        scratch_types=[
            pltpu.VMEM((BPW + 16,), jnp.int32),
            pltpu.VMEM((BPW // GW, GW), jnp.int32),
            pltpu.VMEM((2 * GS, D, GW), jnp.float32),
            pltpu.VMEM((QROWS, GW), jnp.float32),
            pltpu.SemaphoreType.DMA,
            pltpu.SemaphoreType.DMA,
            pltpu.SemaphoreType.DMA,
        ],
    )
    def gather_kernel(u_tab, i_tab, su_h, pu_h, si_h, pi_h, out_u, out_i,
                      idx_v, pos_v, slots, rows, sem_a, sem_b, sem_s):
        wid = lax.axis_index("s") * NUM_CORES + lax.axis_index("c")
        base = wid * BPW
        krow = lax.iota(jnp.int32, 16)
        zc = jnp.int32(0)

        def fire(tab, vec, sbase, sem):
            b0 = vec[0] >> 7
            b1 = vec[1] >> 7
            b2 = vec[2] >> 7
            b3 = vec[3] >> 7
            n1 = (b1 != b0).astype(jnp.int32)
            n2 = (b2 != b1).astype(jnp.int32)
            n3 = (b3 != b2).astype(jnp.int32)
            s0 = sbase
            s1 = s0 + n1
            s2 = s1 + n2
            s3 = s2 + n3
            u_c = 1 + n1 + n2 + n3

            def fire_one(blk_id, slot):
                off = pl.multiple_of(blk_id << 7, GW)
                pltpu.async_copy(
                    tab.at[:, pl.ds(off, GW)], slots.at[slot], sem)

            fire_one(b0, s0)
            pl.when(n1 == 1)(lambda: fire_one(b1, s1))
            pl.when(n2 == 1)(lambda: fire_one(b2, s2))
            pl.when(n3 == 1)(lambda: fire_one(b3, s3))
            return (u_c, vec[0] & 127, vec[1] & 127, vec[2] & 127,
                    vec[3] & 127, s0, s1, s2, s3)

        def extract_one(slot, col, r):
            svec = jnp.full((16,), slot, dtype=jnp.int32)
            cvec = jnp.full((16,), col, dtype=jnp.int32)
            for q in range(D // 16):
                vals = plsc.load_gather(slots, [svec, krow + 16 * q, cvec])
                rows[r, pl.ds(16 * q, 16)] = vals

        def run_quarter(tab, out, h):
            def body(g, carry):
                u_p, c0, c1, c2, c3, s0, s1, s2, s3 = carry
                gg = h * QGRP + g

                def do_fire():
                    vec = idx_v[pl.ds(gg * GS, 16)]
                    sbase = (g % 2) * GS

                    def fa():
                        return fire(tab, vec, sbase, sem_a)

                    def fb():
                        return fire(tab, vec, sbase, sem_b)

                    return lax.cond(g % 2 == 0, fa, fb)

                def no_fire():
                    return (zc, zc, zc, zc, zc, zc, zc, zc, zc)

                new_carry = lax.cond(g < QGRP, do_fire, no_fire)

                @pl.when(g > 0)
                def _drain_extract():
                    def wait_one(_, c):
                        def wa():
                            pltpu.make_async_copy(
                                tab.at[:, pl.ds(0, GW)], slots.at[0],
                                sem_a).wait()
                            return c

                        def wb():
                            pltpu.make_async_copy(
                                tab.at[:, pl.ds(0, GW)], slots.at[0],
                                sem_b).wait()
                            return c

                        return lax.cond(g % 2 == 1, wa, wb)

                    lax.fori_loop(0, u_p, wait_one, 0)
                    r0 = (g - 1) * GS
                    extract_one(s0, c0, r0)
                    extract_one(s1, c1, r0 + 1)
                    extract_one(s2, c2, r0 + 2)
                    extract_one(s3, c3, r0 + 3)

                return new_carry

            lax.fori_loop(0, QGRP + 1, body,
                          (zc, zc, zc, zc, zc, zc, zc, zc, zc))
            pltpu.async_copy(rows, out.at[pos_v.at[h]], sem_s).wait()

        for sidx, spos, tab, out in ((su_h, pu_h, u_tab, out_u),
                                     (si_h, pi_h, i_tab, out_i)):
            pltpu.sync_copy(sidx.at[pl.ds(base, BPW)],
                            idx_v.at[pl.ds(0, BPW)])
            pltpu.sync_copy(spos.at[pl.ds(wid * (BPW // GW), BPW // GW)],
                            pos_v)
            for h in range(4):
                run_quarter(tab, out, h)

    return gather_kernel(u_tab_t, i_tab_t, su, pu2d, si, pi2d)


def _combine_body(u_ref, v_ref, o_ref):
    u = u_ref[...][:, :D]
    v = v_ref[...][:, :D]
    mu = jnp.mean(u, axis=0, keepdims=True)
    mv = jnp.mean(v, axis=0, keepdims=True)
    fu = u + 7.0 * mu
    fv = v + 7.0 * mv
    inter = jnp.sum(fu * fv, axis=1, keepdims=True) * (1.0 / 16.0)
    o_ref[...] = jax.nn.sigmoid(inter)


def kernel(user_indices, item_indices, user_table, item_table):
    iota = lax.iota(jnp.int32, B)
    su, pu = lax.sort_key_val(user_indices, iota)
    si, pi_ = lax.sort_key_val(item_indices, iota)
    u_rows, i_rows = _sc_gather(
        user_table.T, item_table.T,
        su, pu.reshape(B // 128, 128), si, pi_.reshape(B // 128, 128))
    out = pl.pallas_call(
        _combine_body,
        out_shape=jax.ShapeDtypeStruct((B, 1), jnp.float32),
        compiler_params=pltpu.CompilerParams(
            vmem_limit_bytes=100 * 1024 * 1024),
    )(u_rows, i_rows)
    return out.reshape(B)


# speedup vs baseline: 3.2868x; 1.0016x over previous
import functools

import jax
import jax.numpy as jnp
from jax import lax
from jax.experimental import pallas as pl
from jax.experimental.pallas import tpu as pltpu
from jax.experimental.pallas import tpu_sc as plsc

B = 16384
D = 64
NUM_CORES = 2
NUM_SUBCORES = 16
NW = NUM_CORES * NUM_SUBCORES
BPW = B // NW
GW = 128
GS = 4
NGRP = BPW // GS
QGRP = NGRP // 4
QROWS = BPW // 4


def _sc_gather(u_tab_t, i_tab_t, su, pu2d, si, pi2d):
    mesh = plsc.VectorSubcoreMesh(
        core_axis_name="c", subcore_axis_name="s",
        num_cores=NUM_CORES, num_subcores=NUM_SUBCORES,
    )

    @functools.partial(
        pl.kernel,
        out_type=(
            jax.ShapeDtypeStruct((B, GW), jnp.float32),
            jax.ShapeDtypeStruct((B, GW), jnp.float32),
        ),
        mesh=mesh,
        compiler_params=pltpu.CompilerParams(
            use_tc_tiling_on_sc=True, needs_layout_passes=False),
        scratch_types=[
            pltpu.VMEM((BPW + 16,), jnp.int32),
            pltpu.VMEM((BPW // GW, GW), jnp.int32),
            pltpu.VMEM((2 * GS, D, GW), jnp.float32),
            pltpu.VMEM((QROWS, GW), jnp.float32),
            pltpu.SemaphoreType.DMA,
            pltpu.SemaphoreType.DMA,
            pltpu.SemaphoreType.DMA,
        ],
    )
    def gather_kernel(u_tab, i_tab, su_h, pu_h, si_h, pi_h, out_u, out_i,
                      idx_v, pos_v, slots, rows, sem_a, sem_b, sem_s):
        wid = lax.axis_index("s") * NUM_CORES + lax.axis_index("c")
        base = wid * BPW
        krow = lax.iota(jnp.int32, 16)
        zc = jnp.int32(0)

        def fire(tab, vec, sbase, sem):
            b0 = vec[0] >> 7
            b1 = vec[1] >> 7
            b2 = vec[2] >> 7
            b3 = vec[3] >> 7
            n1 = (b1 != b0).astype(jnp.int32)
            n2 = (b2 != b1).astype(jnp.int32)
            n3 = (b3 != b2).astype(jnp.int32)
            s0 = sbase
            s1 = s0 + n1
            s2 = s1 + n2
            s3 = s2 + n3
            u_c = 1 + n1 + n2 + n3

            def fire_one(blk_id, slot):
                off = pl.multiple_of(blk_id << 7, GW)
                pltpu.async_copy(
                    tab.at[:, pl.ds(off, GW)], slots.at[slot], sem)

            fire_one(b0, s0)
            pl.when(n1 == 1)(lambda: fire_one(b1, s1))
            pl.when(n2 == 1)(lambda: fire_one(b2, s2))
            pl.when(n3 == 1)(lambda: fire_one(b3, s3))
            return (u_c, vec[0] & 127, vec[1] & 127, vec[2] & 127,
                    vec[3] & 127, s0, s1, s2, s3)

        def extract_one(slot, col, r):
            svec = jnp.full((16,), slot, dtype=jnp.int32)
            cvec = jnp.full((16,), col, dtype=jnp.int32)
            for q in range(D // 16):
                vals = plsc.load_gather(slots, [svec, krow + 16 * q, cvec])
                rows[r, pl.ds(16 * q, 16)] = vals

        def drain(tab, sem, u_p):
            def wait_one(_, c):
                pltpu.make_async_copy(
                    tab.at[:, pl.ds(0, GW)], slots.at[0], sem).wait()
                return c

            lax.fori_loop(0, u_p, wait_one, 0)

        def extract_grp(st, r0):
            _, c0, c1, c2, c3, s0, s1, s2, s3 = st
            extract_one(s0, c0, r0)
            extract_one(s1, c1, r0 + 1)
            extract_one(s2, c2, r0 + 2)
            extract_one(s3, c3, r0 + 3)

        def run_quarter(tab, out, h):
            def body(t, carry):
                g_even = 2 * t
                g_odd = 2 * t + 1
                gg0 = h * QGRP + g_even
                st_a = fire(tab, idx_v[pl.ds(gg0 * GS, 16)], 0, sem_a)

                @pl.when(t > 0)
                def _db():
                    drain(tab, sem_b, carry[0])
                    extract_grp(carry, (g_even - 1) * GS)

                st_b = fire(tab, idx_v[pl.ds((gg0 + 1) * GS, 16)], GS,
                            sem_b)
                drain(tab, sem_a, st_a[0])
                extract_grp(st_a, g_even * GS)
                return st_b

            last = lax.fori_loop(
                0, QGRP // 2, body,
                (zc, zc, zc, zc, zc, zc, zc, zc, zc))
            drain(tab, sem_b, last[0])
            extract_grp(last, (QGRP - 1) * GS)
            pltpu.async_copy(rows, out.at[pos_v.at[h]], sem_s).wait()

        for sidx, spos, tab, out in ((su_h, pu_h, u_tab, out_u),
                                     (si_h, pi_h, i_tab, out_i)):
            pltpu.sync_copy(sidx.at[pl.ds(base, BPW)],
                            idx_v.at[pl.ds(0, BPW)])
            pltpu.sync_copy(spos.at[pl.ds(wid * (BPW // GW), BPW // GW)],
                            pos_v)
            for h in range(4):
                run_quarter(tab, out, h)

    return gather_kernel(u_tab_t, i_tab_t, su, pu2d, si, pi2d)


def _combine_body(u_ref, v_ref, o_ref):
    u = u_ref[...][:, :D]
    v = v_ref[...][:, :D]
    mu = jnp.mean(u, axis=0, keepdims=True)
    mv = jnp.mean(v, axis=0, keepdims=True)
    fu = u + 7.0 * mu
    fv = v + 7.0 * mv
    inter = jnp.sum(fu * fv, axis=1, keepdims=True) * (1.0 / 16.0)
    o_ref[...] = jax.nn.sigmoid(inter)


def kernel(user_indices, item_indices, user_table, item_table):
    iota = lax.iota(jnp.int32, B)
    su, pu = lax.sort_key_val(user_indices, iota)
    si, pi_ = lax.sort_key_val(item_indices, iota)
    u_rows, i_rows = _sc_gather(
        user_table.T, item_table.T,
        su, pu.reshape(B // 128, 128), si, pi_.reshape(B // 128, 128))
    out = pl.pallas_call(
        _combine_body,
        out_shape=jax.ShapeDtypeStruct((B, 1), jnp.float32),
        compiler_params=pltpu.CompilerParams(
            vmem_limit_bytes=100 * 1024 * 1024),
    )(u_rows, i_rows)
    return out.reshape(B)
